# Initial kernel scaffold; baseline (speedup 1.0000x reference)
import jax
import jax.numpy as jnp
from jax.experimental import pallas as pl


def kernel(input, action_embedding):
    raise NotImplementedError("write your pallas kernel here")


# speedup vs baseline: 1.0948x; 1.0948x over previous
import functools

import jax
import jax.numpy as jnp
from jax import lax
from jax.experimental import pallas as pl
from jax.experimental.pallas import tpu as pltpu
from jax.experimental.pallas import tpu_sc as plsc

NUM_ACTIONS = 1000000
LATENT_DIM = 32
BATCH = 16384
HIST = 50

_NC = 2
_NS = 16
_NW = _NC * _NS

_B = BATCH * HIST
_BPW = _B // _NW
_CHUNK = 1024
_NCHUNK = _BPW // _CHUNK


def _gather_kernel(table_hbm, idx_hbm, out_hbm, idx_v, rows_v, sem):
    wid = lax.axis_index("s") * _NC + lax.axis_index("c")
    base = wid * _BPW

    def chunk_body(i, carry):
        off = base + i * _CHUNK
        pltpu.sync_copy(idx_hbm.at[pl.ds(off, _CHUNK)], idx_v)
        pltpu.async_copy(table_hbm.at[idx_v], rows_v, sem).wait()
        pltpu.sync_copy(rows_v, out_hbm.at[pl.ds(off, _CHUNK)])
        return carry

    lax.fori_loop(0, _NCHUNK, chunk_body, 0)


@jax.jit
def _embed_lookup(idx_flat, table):
    mesh = plsc.VectorSubcoreMesh(core_axis_name="c", subcore_axis_name="s")
    kfn = functools.partial(
        pl.kernel,
        mesh=mesh,
        out_type=jax.ShapeDtypeStruct((_B, LATENT_DIM), jnp.float32),
        scratch_types=[
            pltpu.VMEM((_CHUNK,), jnp.int32),
            pltpu.VMEM((_CHUNK, LATENT_DIM), jnp.float32),
            pltpu.SemaphoreType.DMA,
        ],
        compiler_params=pltpu.CompilerParams(use_tc_tiling_on_sc=False),
    )(_gather_kernel)
    return kfn(table, idx_flat)


def kernel(input, action_embedding):
    idx_flat = input.astype(jnp.int32).reshape(-1)
    out = _embed_lookup(idx_flat, action_embedding)
    return out.reshape(BATCH, HIST, LATENT_DIM)


# speedup vs baseline: 1.1091x; 1.0131x over previous
import functools

import jax
import jax.numpy as jnp
from jax import lax
from jax.experimental import pallas as pl
from jax.experimental.pallas import tpu as pltpu
from jax.experimental.pallas import tpu_sc as plsc

NUM_ACTIONS = 1000000
LATENT_DIM = 32
BATCH = 16384
HIST = 50

_NC = 2
_NS = 16
_NW = _NC * _NS

_B = BATCH * HIST
_BPW = _B // _NW
_CHUNK = 1600
_NCHUNK = _BPW // _CHUNK


def _gather_kernel(table_hbm, idx_hbm, out_hbm,
                   idx0, idx1, rows0, rows1,
                   si0, si1, sg0, sg1, so0, so1):
    wid = lax.axis_index("s") * _NC + lax.axis_index("c")
    base = wid * _BPW

    idx_v = (idx0, idx1)
    rows_v = (rows0, rows1)
    si = (si0, si1)
    sg = (sg0, sg1)
    so = (so0, so1)

    def idx_start(i):
        off = base + i * _CHUNK
        pltpu.async_copy(idx_hbm.at[pl.ds(off, _CHUNK)], idx_v[i % 2],
                         si[i % 2])

    idx_start(0)
    idx_start(1)

    for i in range(_NCHUNK):
        s = i % 2
        pltpu.make_async_copy(
            idx_hbm.at[pl.ds(base + i * _CHUNK, _CHUNK)], idx_v[s], si[s]
        ).wait()
        if i >= 2:
            pltpu.make_async_copy(
                rows_v[s], out_hbm.at[pl.ds(base + (i - 2) * _CHUNK, _CHUNK)],
                so[s]).wait()
        pltpu.async_copy(table_hbm.at[idx_v[s]], rows_v[s], sg[s]).wait()
        pltpu.async_copy(rows_v[s],
                         out_hbm.at[pl.ds(base + i * _CHUNK, _CHUNK)], so[s])
        if i + 2 < _NCHUNK:
            idx_start(i + 2)

    for i in (_NCHUNK - 2, _NCHUNK - 1):
        s = i % 2
        pltpu.make_async_copy(
            rows_v[s], out_hbm.at[pl.ds(base + i * _CHUNK, _CHUNK)], so[s]
        ).wait()


@jax.jit
def _embed_lookup(idx_flat, table):
    mesh = plsc.VectorSubcoreMesh(core_axis_name="c", subcore_axis_name="s")
    kfn = functools.partial(
        pl.kernel,
        mesh=mesh,
        out_type=jax.ShapeDtypeStruct((_B, LATENT_DIM), jnp.float32),
        scratch_types=[
            pltpu.VMEM((_CHUNK,), jnp.int32),
            pltpu.VMEM((_CHUNK,), jnp.int32),
            pltpu.VMEM((_CHUNK, LATENT_DIM), jnp.float32),
            pltpu.VMEM((_CHUNK, LATENT_DIM), jnp.float32),
            pltpu.SemaphoreType.DMA,
            pltpu.SemaphoreType.DMA,
            pltpu.SemaphoreType.DMA,
            pltpu.SemaphoreType.DMA,
            pltpu.SemaphoreType.DMA,
            pltpu.SemaphoreType.DMA,
        ],
        compiler_params=pltpu.CompilerParams(use_tc_tiling_on_sc=False),
    )(_gather_kernel)
    return kfn(table, idx_flat)


def kernel(input, action_embedding):
    idx_flat = input.astype(jnp.int32).reshape(-1)
    out = _embed_lookup(idx_flat, action_embedding)
    return out.reshape(BATCH, HIST, LATENT_DIM)


# speedup vs baseline: 1.7885x; 1.6125x over previous
import functools

import jax
import jax.numpy as jnp
from jax import lax
from jax.experimental import pallas as pl
from jax.experimental.pallas import tpu as pltpu
from jax.experimental.pallas import tpu_sc as plsc

NUM_ACTIONS = 1000000
LATENT_DIM = 32
BATCH = 16384
HIST = 50

_NC = 2
_NS = 16
_NW = _NC * _NS

_B = BATCH * HIST
_BPW = _B // _NW
_CHUNK = 1600
_NROWS = _CHUNK // HIST
_NCHUNK = _BPW // _CHUNK


def _gather_kernel(table_hbm, idx_hbm, out_hbm,
                   idx0, idx1, rows0, rows1,
                   si0, si1, sg0, sg1, so0, so1):
    wid = lax.axis_index("s") * _NC + lax.axis_index("c")
    base = wid * _BPW

    idx_v = (idx0, idx1)
    rows_v = (rows0, rows1)
    si = (si0, si1)
    sg = (sg0, sg1)
    so = (so0, so1)

    def idx_start(i):
        off = base + i * _CHUNK
        pltpu.async_copy(idx_hbm.at[pl.ds(off, _CHUNK)], idx_v[i % 2],
                         si[i % 2])

    def out_start(i, s):
        b_off = (base + i * _CHUNK) // HIST

        def row_body(r, carry):
            pltpu.async_copy(rows_v[s].at[pl.ds(r * HIST, HIST)],
                             out_hbm.at[b_off + r], so[s])
            return carry

        lax.fori_loop(0, _NROWS, row_body, 0)

    def out_wait(i, s):
        b_off = (base + i * _CHUNK) // HIST

        def row_body(r, carry):
            pltpu.make_async_copy(rows_v[s].at[pl.ds(r * HIST, HIST)],
                                  out_hbm.at[b_off + r], so[s]).wait()
            return carry

        lax.fori_loop(0, _NROWS, row_body, 0)

    idx_start(0)
    idx_start(1)

    for i in range(_NCHUNK):
        s = i % 2
        pltpu.make_async_copy(
            idx_hbm.at[pl.ds(base + i * _CHUNK, _CHUNK)], idx_v[s], si[s]
        ).wait()
        if i >= 2:
            out_wait(i - 2, s)
        pltpu.async_copy(table_hbm.at[idx_v[s]], rows_v[s], sg[s]).wait()
        out_start(i, s)
        if i + 2 < _NCHUNK:
            idx_start(i + 2)

    for i in (_NCHUNK - 2, _NCHUNK - 1):
        out_wait(i, i % 2)


@jax.jit
def _embed_lookup(idx_flat, table):
    mesh = plsc.VectorSubcoreMesh(core_axis_name="c", subcore_axis_name="s")
    kfn = functools.partial(
        pl.kernel,
        mesh=mesh,
        out_type=jax.ShapeDtypeStruct((BATCH, HIST, LATENT_DIM), jnp.float32),
        scratch_types=[
            pltpu.VMEM((_CHUNK,), jnp.int32),
            pltpu.VMEM((_CHUNK,), jnp.int32),
            pltpu.VMEM((_CHUNK, LATENT_DIM), jnp.float32),
            pltpu.VMEM((_CHUNK, LATENT_DIM), jnp.float32),
            pltpu.SemaphoreType.DMA,
            pltpu.SemaphoreType.DMA,
            pltpu.SemaphoreType.DMA,
            pltpu.SemaphoreType.DMA,
            pltpu.SemaphoreType.DMA,
            pltpu.SemaphoreType.DMA,
        ],
        compiler_params=pltpu.CompilerParams(use_tc_tiling_on_sc=False),
    )(_gather_kernel)
    return kfn(table, idx_flat)


def kernel(input, action_embedding):
    idx_flat = input.astype(jnp.int32).reshape(-1)
    return _embed_lookup(idx_flat, action_embedding)
